# Initial kernel scaffold; baseline (speedup 1.0000x reference)
import jax
import jax.numpy as jnp
from jax.experimental import pallas as pl


def kernel(X, B, L_index, L_values, Lu_index, Lu_values, Ld_index, Ld_values, W1_irr, W1_up, W1_down, W2_irr, W2_up, W2_down, W3_irr, W3_up, W3_down):
    raise NotImplementedError("write your pallas kernel here")


# speedup vs baseline: 3.3400x; 3.3400x over previous
import functools

import jax
import jax.numpy as jnp
from jax import lax
from jax.experimental import pallas as pl
from jax.experimental.pallas import tpu as pltpu
from jax.experimental.pallas import tpu_sc as plsc

N = 10000
E = 320000
D = 128

NC = 2
NS = 16
DH = D // NC
K = 80
CPT = E // K // NS
REG = 400
NREG = N // REG

MB = 400
GRID = N // MB


def _mm3_body(x_ref, wi_ref, wu_ref, wd_ref, yi_ref, yu_ref, yd_ref):
    h = x_ref[...]
    yi_ref[...] = jnp.dot(h, wi_ref[...], preferred_element_type=jnp.float32)
    yu_ref[...] = jnp.dot(h, wu_ref[...], preferred_element_type=jnp.float32)
    yd_ref[...] = jnp.dot(h, wd_ref[...], preferred_element_type=jnp.float32)


def _tanh_mm3_body(p_ref, wi_ref, wu_ref, wd_ref, yi_ref, yu_ref, yd_ref):
    h = jnp.tanh(jnp.concatenate([p_ref[0], p_ref[1]], axis=-1))
    yi_ref[...] = jnp.dot(h, wi_ref[...], preferred_element_type=jnp.float32)
    yu_ref[...] = jnp.dot(h, wu_ref[...], preferred_element_type=jnp.float32)
    yd_ref[...] = jnp.dot(h, wd_ref[...], preferred_element_type=jnp.float32)


def _tanh_sum_body(p_ref, o_ref):
    o_ref[...] = jnp.tanh(jnp.concatenate([p_ref[0], p_ref[1]], axis=-1))


_w_spec = pl.BlockSpec((D, D), lambda i: (0, 0))
_x_spec = pl.BlockSpec((MB, D), lambda i: (i, 0))
_p_spec = pl.BlockSpec((NC, MB, DH), lambda i: (0, i, 0))
_y_out = [jax.ShapeDtypeStruct((N, D), jnp.float32)] * 3

_mm3 = pl.pallas_call(
    _mm3_body,
    grid=(GRID,),
    in_specs=[_x_spec, _w_spec, _w_spec, _w_spec],
    out_specs=[_x_spec, _x_spec, _x_spec],
    out_shape=_y_out,
)

_tanh_mm3 = pl.pallas_call(
    _tanh_mm3_body,
    grid=(GRID,),
    in_specs=[_p_spec, _w_spec, _w_spec, _w_spec],
    out_specs=[_x_spec, _x_spec, _x_spec],
    out_shape=_y_out,
)

_tanh_sum = pl.pallas_call(
    _tanh_sum_body,
    grid=(GRID,),
    in_specs=[_p_spec],
    out_specs=_x_spec,
    out_shape=jax.ShapeDtypeStruct((N, D), jnp.float32),
)


def _sc_body(yi, yu, yd,
             src_i, dst_i, val_i, src_u, dst_u, val_u, src_d, dst_d, val_d,
             out, src_v, dst_v, val_v, rows_v, acc_sh, sem):
    c = lax.axis_index("c")
    s = lax.axis_index("s")
    zeros16 = jnp.zeros((16,), jnp.float32)

    def zrow(r, carry):
        for cb in range(DH // 16):
            rows_v[r, pl.ds(cb * 16, 16)] = zeros16
        return carry
    lax.fori_loop(0, K, zrow, 0)

    def zero_region(r0):
        for i in range(REG // K):
            pltpu.sync_copy(rows_v, acc_sh.at[pl.ds(r0 + i * K, K), :])

    zero_region(s * REG)

    @pl.when(s + NS < NREG)
    def _():
        zero_region((s + NS) * REG)

    plsc.subcore_barrier()

    for y_hbm, src_hbm, dst_hbm, val_hbm in (
            (yi, src_i, dst_i, val_i),
            (yu, src_u, dst_u, val_u),
            (yd, src_d, dst_d, val_d)):
        pltpu.sync_copy(src_hbm.at[s], src_v)
        pltpu.sync_copy(dst_hbm.at[s], dst_v)
        pltpu.sync_copy(val_hbm.at[s], val_v)

        def chunk(j, carry):
            for g in range(K // 16):
                sl = pl.ds(g * 16, 16)
                src_v[j, sl] = src_v[j, sl] * 2 + jnp.broadcast_to(c, (16,))
            pltpu.async_copy(y_hbm.at[src_v.at[j]], rows_v, sem).wait()

            def grp(g, carry2):
                vals16 = val_v[j, pl.ds(g * 16, 16)]
                for jj in range(16):
                    vb = jnp.broadcast_to(vals16[jj], (16,))
                    r = g * 16 + jj
                    for cb in range(DH // 16):
                        sl = pl.ds(cb * 16, 16)
                        rows_v[r, sl] = rows_v[r, sl] * vb
                return carry2
            lax.fori_loop(0, K // 16, grp, 0)

            pltpu.sync_copy(rows_v, acc_sh.at[dst_v.at[j]], add=True)
            return carry
        lax.fori_loop(0, CPT, chunk, 0)

    plsc.subcore_barrier()

    def dump_region(r0):
        pltpu.sync_copy(acc_sh.at[pl.ds(r0, REG), :],
                        out.at[c, pl.ds(r0, REG), :])

    dump_region(s * REG)

    @pl.when(s + NS < NREG)
    def _():
        dump_region((s + NS) * REG)


_sc_spmm = pl.kernel(
    _sc_body,
    out_type=jax.ShapeDtypeStruct((NC, N, DH), jnp.float32),
    mesh=plsc.VectorSubcoreMesh(core_axis_name="c", subcore_axis_name="s"),
    compiler_params=pltpu.CompilerParams(use_tc_tiling_on_sc=False),
    scratch_types=[
        pltpu.VMEM((CPT, K), jnp.int32),
        pltpu.VMEM((CPT, K), jnp.int32),
        pltpu.VMEM((CPT, K), jnp.float32),
        pltpu.VMEM((K, DH), jnp.float32),
        pltpu.VMEM_SHARED((N, DH), jnp.float32),
        pltpu.SemaphoreType.DMA,
    ],
)


def kernel(X, B, L_index, L_values, Lu_index, Lu_values, Ld_index, Ld_values,
           W1_irr, W1_up, W1_down, W2_irr, W2_up, W2_down,
           W3_irr, W3_up, W3_down):
    del B
    def prep(idx, vals):
        shp = (NS, CPT, K)
        src = idx[0].astype(jnp.int32).reshape(shp)
        dst = idx[1].astype(jnp.int32).reshape(shp)
        return src, dst, vals.reshape(shp)

    si, di, vi = prep(L_index, L_values)
    su, du, vu = prep(Lu_index, Lu_values)
    sd, dd, vd = prep(Ld_index, Ld_values)

    def spmm(ys):
        y2 = [y.reshape(NC * N, DH) for y in ys]
        return _sc_spmm(y2[0], y2[1], y2[2], si, di, vi, su, du, vu, sd, dd, vd)

    p = spmm(_mm3(X, W1_irr, W1_up, W1_down))
    p = spmm(_tanh_mm3(p, W2_irr, W2_up, W2_down))
    p = spmm(_tanh_mm3(p, W3_irr, W3_up, W3_down))
    return _tanh_sum(p)
